# baseline (speedup 1.0000x reference)
import functools

import jax
import jax.numpy as jnp
from jax import lax
from jax.experimental import pallas as pl
from jax.experimental.pallas import tpu as pltpu
from jax.experimental.pallas import tpu_sc as plsc

N = 10000
E = 320000
H = 128
DE = 16
COORD_SCALE = 0.1

NC = 2
NS = 16
L = 16
NW = NC * NS
EW = E // NW
C = 80
NCH = EW // C
N2 = 10240
NPT = N2 // NS
ZR = 128
TW = 16

@functools.cache
def _mesh():
    return plsc.VectorSubcoreMesh(core_axis_name="c", subcore_axis_name="s",
                                  num_cores=NC, num_subcores=NS)


@functools.cache
def _sc_gather_kernel():
  return pl.kernel(
    _sc_gather,
    out_type=(
        jax.ShapeDtypeStruct((E, H), jnp.bfloat16),
        jax.ShapeDtypeStruct((E, H), jnp.bfloat16),
        jax.ShapeDtypeStruct((E, 4), jnp.float32),
    ),
    mesh=_mesh(),
    scratch_types=[
        pltpu.VMEM((N,), jnp.float32),
        pltpu.VMEM((N,), jnp.float32),
        pltpu.VMEM((N,), jnp.float32),
        pltpu.VMEM((C,), jnp.int32),
        pltpu.VMEM((C,), jnp.int32),
        pltpu.VMEM((C, H), jnp.bfloat16),
        pltpu.VMEM((C, H), jnp.bfloat16),
        pltpu.VMEM((C, 4), jnp.float32),
        pltpu.SemaphoreType.DMA,
        pltpu.SemaphoreType.DMA,
    ],
    compiler_params=pltpu.CompilerParams(needs_layout_passes=False,
                                         use_tc_tiling_on_sc=False),
  )


def _sc_gather(h_hbm, px_hbm, py_hbm, pz_hbm, src_hbm, dst_hbm, hs_out, hd_out, geom_out,
               px_v, py_v, pz_v, is_v, id_v, hs_v, hd_v, geom_v, sem_s, sem_d):
    cid = lax.axis_index("c")
    sid = lax.axis_index("s")
    wid = sid * NC + cid

    pltpu.sync_copy(px_hbm, px_v)
    pltpu.sync_copy(py_hbm, py_v)
    pltpu.sync_copy(pz_hbm, pz_v)

    def body(i, carry):
        base = wid * EW + i * C
        pltpu.sync_copy(src_hbm.at[pl.ds(base, C)], is_v)
        pltpu.sync_copy(dst_hbm.at[pl.ds(base, C)], id_v)
        cp_s = pltpu.async_copy(h_hbm.at[is_v], hs_v, sem_s)
        cp_d = pltpu.async_copy(h_hbm.at[id_v], hd_v, sem_d)
        for j in range(C // L):
            rs = is_v[pl.ds(j * L, L)]
            rd = id_v[pl.ds(j * L, L)]
            dx = plsc.load_gather(px_v, [rd]) - plsc.load_gather(px_v, [rs])
            dy = plsc.load_gather(py_v, [rd]) - plsc.load_gather(py_v, [rs])
            dz = plsc.load_gather(pz_v, [rd]) - plsc.load_gather(pz_v, [rs])
            d2 = jnp.minimum(dx * dx + dy * dy + dz * dz, 1000.0)
            rows = j * L + lax.iota(jnp.int32, L)
            plsc.store_scatter(geom_v, [rows, jnp.full((L,), 0, jnp.int32)], dx)
            plsc.store_scatter(geom_v, [rows, jnp.full((L,), 1, jnp.int32)], dy)
            plsc.store_scatter(geom_v, [rows, jnp.full((L,), 2, jnp.int32)], dz)
            plsc.store_scatter(geom_v, [rows, jnp.full((L,), 3, jnp.int32)], d2)
        cp_s.wait()
        cp_d.wait()
        pltpu.sync_copy(hs_v, hs_out.at[pl.ds(base, C)])
        pltpu.sync_copy(hd_v, hd_out.at[pl.ds(base, C)])
        pltpu.sync_copy(geom_v, geom_out.at[pl.ds(base, C)])
        return carry

    lax.fori_loop(0, NCH, body, 0)


@functools.cache
def _sc_scatter_kernel():
  return pl.kernel(
    _sc_scatter,
    out_type=(
        jax.ShapeDtypeStruct((NC, N2, H), jnp.float32),
        jax.ShapeDtypeStruct((NC, N2, TW), jnp.float32),
    ),
    mesh=_mesh(),
    scratch_types=[
        pltpu.VMEM_SHARED((N2, H), jnp.float32),
        pltpu.VMEM_SHARED((N2, TW), jnp.float32),
        pltpu.VMEM((C, H), jnp.float32),
        pltpu.VMEM((C, TW), jnp.float32),
        pltpu.VMEM((C,), jnp.int32),
        pltpu.VMEM((ZR, H), jnp.float32),
        pltpu.VMEM((NPT, TW), jnp.float32),
    ],
    compiler_params=pltpu.CompilerParams(needs_layout_passes=False,
                                         use_tc_tiling_on_sc=False),
  )


def _sc_scatter(m_hbm, t_hbm, dst_hbm, agg_out, dpos_out,
                agg_sh, dpos_sh, m_v, t_v, id_v, zb_v, zbt_v):
    cid = lax.axis_index("c")
    sid = lax.axis_index("s")
    wid = sid * NC + cid
    r0 = sid * NPT

    zero = jnp.zeros((L,), jnp.float32)

    def zrow(r, carry):
        for j in range(H // L):
            zb_v[r, pl.ds(j * L, L)] = zero
        return carry

    lax.fori_loop(0, ZR, zrow, 0)

    def zrow_t(r, carry):
        zbt_v[r] = zero
        return carry

    lax.fori_loop(0, NPT, zrow_t, 0)

    for k in range(NPT // ZR):
        pltpu.sync_copy(zb_v, agg_sh.at[pl.ds(r0 + k * ZR, ZR)])
    pltpu.sync_copy(zbt_v, dpos_sh.at[pl.ds(r0, NPT)])
    plsc.subcore_barrier()

    def body(i, carry):
        base = wid * EW + i * C
        pltpu.sync_copy(dst_hbm.at[pl.ds(base, C)], id_v)
        pltpu.sync_copy(m_hbm.at[pl.ds(base, C)], m_v)
        pltpu.sync_copy(t_hbm.at[pl.ds(base, C)], t_v)
        pltpu.sync_copy(m_v, agg_sh.at[id_v], add=True)
        pltpu.sync_copy(t_v, dpos_sh.at[id_v], add=True)
        return carry

    lax.fori_loop(0, NCH, body, 0)
    plsc.subcore_barrier()

    pltpu.sync_copy(agg_sh.at[pl.ds(r0, NPT)], agg_out.at[cid, pl.ds(r0, NPT)])
    pltpu.sync_copy(dpos_sh.at[pl.ds(r0, NPT)], dpos_out.at[cid, pl.ds(r0, NPT)])


def _edge_mlp_body(hs_ref, hd_ref, ea_ref, g_ref,
                   w1s_ref, w1d_ref, w1e_ref, w1g_ref, b1_ref,
                   w2_ref, b2_ref, wc1_ref, bc1_ref, wc2_ref, bc2_ref,
                   m_ref, t_ref):
    f32 = jnp.float32
    bf16 = jnp.bfloat16
    t1 = jnp.dot(hs_ref[...], w1s_ref[...], preferred_element_type=f32)
    t1 += jnp.dot(hd_ref[...], w1d_ref[...], preferred_element_type=f32)
    t1 += jnp.dot(ea_ref[...], w1e_ref[...], preferred_element_type=f32)
    d2 = g_ref[:, 3:4]
    t1 += d2 * w1g_ref[...] + b1_ref[...]
    t1 = jnp.maximum(t1, 0.0)
    m = jnp.maximum(jnp.dot(t1.astype(bf16), w2_ref[...],
                            preferred_element_type=f32) + b2_ref[...], 0.0)
    c = jnp.maximum(jnp.dot(m.astype(bf16), wc1_ref[...],
                            preferred_element_type=f32) + bc1_ref[...], 0.0)
    s = jnp.sum(c * wc2_ref[...], axis=1, keepdims=True) + bc2_ref[...]
    coef = jnp.tanh(s) * COORD_SCALE
    inv = lax.rsqrt(d2 + 1e-8)
    dirs = g_ref[:, 0:3] * inv
    m_ref[...] = m
    t_ref[...] = jnp.pad(dirs * coef, ((0, 0), (0, TW - 3)))


def _edge_mlp(hs, hd, ea, geom, w1s, w1d, w1e, w1g, b1, w2, b2, wc1, bc1, wc2, bc2):
    B = 512
    grid = (E // B,)
    full = lambda shape: pl.BlockSpec(shape, lambda i: (0,) * len(shape))
    row = lambda width: pl.BlockSpec((B, width), lambda i: (i, 0))
    return pl.pallas_call(
        _edge_mlp_body,
        grid=grid,
        in_specs=[row(H), row(H), row(DE), row(4),
                  full((H, H)), full((H, H)), full((DE, H)), full((1, H)),
                  full((1, H)), full((H, H)), full((1, H)), full((H, H)),
                  full((1, H)), full((1, H)), full((1, 1))],
        out_specs=[row(H), row(TW)],
        out_shape=[jax.ShapeDtypeStruct((E, H), jnp.float32),
                   jax.ShapeDtypeStruct((E, TW), jnp.float32)],
    )(hs, hd, ea, geom, w1s, w1d, w1e, w1g, b1, w2, b2, wc1, bc1, wc2, bc2)


def _node_body(h_ref, pos_ref, agg_ref, dpos_ref,
               wnh_ref, wna_ref, bn_ref, g_ref, b_ref, ho_ref, po_ref):
    f32 = jnp.float32
    h = h_ref[...]
    agg = agg_ref[0] + agg_ref[1]
    u = jnp.dot(h, wnh_ref[...], preferred_element_type=f32)
    u += jnp.dot(agg, wna_ref[...], preferred_element_type=f32)
    u = jnp.maximum(u + bn_ref[...], 0.0)
    y = h + u
    mu = jnp.mean(y, axis=1, keepdims=True)
    yc = y - mu
    var = jnp.mean(yc * yc, axis=1, keepdims=True)
    ho_ref[...] = yc * lax.rsqrt(var + 1e-5) * g_ref[...] + b_ref[...]
    dp = dpos_ref[0, :, 0:3] + dpos_ref[1, :, 0:3]
    po_ref[...] = pos_ref[...] + dp


def _node_update(h, pos, aggp, dposp, wnh, wna, bn, gamma, beta):
    B = 1000
    grid = (N // B,)
    full = lambda shape: pl.BlockSpec(shape, lambda i: (0,) * len(shape))
    return pl.pallas_call(
        _node_body,
        grid=grid,
        in_specs=[pl.BlockSpec((B, H), lambda i: (i, 0)),
                  pl.BlockSpec((B, 3), lambda i: (i, 0)),
                  pl.BlockSpec((NC, B, H), lambda i: (0, i, 0)),
                  pl.BlockSpec((NC, B, TW), lambda i: (0, i, 0)),
                  full((H, H)), full((H, H)), full((1, H)),
                  full((1, H)), full((1, H))],
        out_specs=[pl.BlockSpec((B, H), lambda i: (i, 0)),
                   pl.BlockSpec((B, 3), lambda i: (i, 0))],
        out_shape=[jax.ShapeDtypeStruct((N, H), jnp.float32),
                   jax.ShapeDtypeStruct((N, 3), jnp.float32)],
    )(h, pos, aggp, dposp, wnh, wna, bn, gamma, beta)


def kernel(h, pos, edge_attr, We1, be1, We2, be2, Wc1, bc1, Wc2, bc2,
           Wn, bn, gamma, beta, edge_index):
    src = edge_index[0]
    dst = edge_index[1]

    hb = h.astype(jnp.bfloat16)
    hs, hd, geom = _sc_gather_kernel()(hb, pos[:, 0], pos[:, 1], pos[:, 2],
                                       src, dst)

    w1s = We1[:, :H].T
    w1d = We1[:, H:2 * H].T
    w1g = We1[:, 2 * H:2 * H + 1].T
    w1e = We1[:, 2 * H + 1:].T
    bf16 = jnp.bfloat16
    m, trans = _edge_mlp(hs, hd, edge_attr.astype(bf16), geom,
                         w1s.astype(bf16), w1d.astype(bf16),
                         w1e.astype(bf16), w1g, be1.reshape(1, H),
                         We2.T.astype(bf16), be2.reshape(1, H),
                         Wc1.T.astype(bf16), bc1.reshape(1, H),
                         Wc2.reshape(1, H), bc2.reshape(1, 1))

    aggp, dposp = _sc_scatter_kernel()(m, trans, dst)
    aggp = aggp[:, :N]
    dposp = dposp[:, :N]

    h_out, pos_out = _node_update(h, pos, aggp, dposp,
                                  Wn[:, :H].T, Wn[:, H:].T, bn.reshape(1, H),
                                  gamma.reshape(1, H), beta.reshape(1, H))
    return (h_out, pos_out)


# speedup vs baseline: 1.3091x; 1.3091x over previous
import functools

import jax
import jax.numpy as jnp
from jax import lax
from jax.experimental import pallas as pl
from jax.experimental.pallas import tpu as pltpu
from jax.experimental.pallas import tpu_sc as plsc

N = 10000
E = 320000
H = 128
DE = 16
COORD_SCALE = 0.1

NC = 2
NS = 16
L = 16
NW = NC * NS
EW = E // NW
C = 80
NCH = EW // C
N2 = 10240
NPT = N2 // NS
ZR = 128
TW = 16

@functools.cache
def _mesh():
    return plsc.VectorSubcoreMesh(core_axis_name="c", subcore_axis_name="s",
                                  num_cores=NC, num_subcores=NS)


@functools.cache
def _sc_gather_kernel():
  return pl.kernel(
    _sc_gather,
    out_type=(
        jax.ShapeDtypeStruct((E, H), jnp.float32),
        jax.ShapeDtypeStruct((E, H), jnp.float32),
        jax.ShapeDtypeStruct((E, 4), jnp.float32),
    ),
    mesh=_mesh(),
    scratch_types=[
        pltpu.VMEM((N,), jnp.float32),
        pltpu.VMEM((N,), jnp.float32),
        pltpu.VMEM((N,), jnp.float32),
        pltpu.VMEM((C,), jnp.int32),
        pltpu.VMEM((C,), jnp.int32),
        pltpu.VMEM((C, H), jnp.float32),
        pltpu.VMEM((C, H), jnp.float32),
        pltpu.VMEM((C, 4), jnp.float32),
        pltpu.SemaphoreType.DMA,
        pltpu.SemaphoreType.DMA,
    ],
    compiler_params=pltpu.CompilerParams(needs_layout_passes=False,
                                         use_tc_tiling_on_sc=False),
  )


def _sc_gather(h_hbm, px_hbm, py_hbm, pz_hbm, src_hbm, dst_hbm, hs_out, hd_out, geom_out,
               px_v, py_v, pz_v, is_v, id_v, hs_v, hd_v, geom_v, sem_s, sem_d):
    cid = lax.axis_index("c")
    sid = lax.axis_index("s")
    wid = sid * NC + cid

    pltpu.sync_copy(px_hbm, px_v)
    pltpu.sync_copy(py_hbm, py_v)
    pltpu.sync_copy(pz_hbm, pz_v)

    def body(i, carry):
        base = wid * EW + i * C
        pltpu.sync_copy(src_hbm.at[pl.ds(base, C)], is_v)
        pltpu.sync_copy(dst_hbm.at[pl.ds(base, C)], id_v)
        cp_s = pltpu.async_copy(h_hbm.at[is_v], hs_v, sem_s)
        cp_d = pltpu.async_copy(h_hbm.at[id_v], hd_v, sem_d)
        for j in range(C // L):
            rs = is_v[pl.ds(j * L, L)]
            rd = id_v[pl.ds(j * L, L)]
            dx = plsc.load_gather(px_v, [rd]) - plsc.load_gather(px_v, [rs])
            dy = plsc.load_gather(py_v, [rd]) - plsc.load_gather(py_v, [rs])
            dz = plsc.load_gather(pz_v, [rd]) - plsc.load_gather(pz_v, [rs])
            d2 = jnp.minimum(dx * dx + dy * dy + dz * dz, 1000.0)
            rows = j * L + lax.iota(jnp.int32, L)
            plsc.store_scatter(geom_v, [rows, jnp.full((L,), 0, jnp.int32)], dx)
            plsc.store_scatter(geom_v, [rows, jnp.full((L,), 1, jnp.int32)], dy)
            plsc.store_scatter(geom_v, [rows, jnp.full((L,), 2, jnp.int32)], dz)
            plsc.store_scatter(geom_v, [rows, jnp.full((L,), 3, jnp.int32)], d2)
        cp_s.wait()
        cp_d.wait()
        pltpu.sync_copy(hs_v, hs_out.at[pl.ds(base, C)])
        pltpu.sync_copy(hd_v, hd_out.at[pl.ds(base, C)])
        pltpu.sync_copy(geom_v, geom_out.at[pl.ds(base, C)])
        return carry

    lax.fori_loop(0, NCH, body, 0)


@functools.cache
def _sc_scatter_kernel():
  return pl.kernel(
    _sc_scatter,
    out_type=(
        jax.ShapeDtypeStruct((NC, N2, H), jnp.float32),
        jax.ShapeDtypeStruct((NC, N2, TW), jnp.float32),
    ),
    mesh=_mesh(),
    scratch_types=[
        pltpu.VMEM_SHARED((N2, H), jnp.float32),
        pltpu.VMEM_SHARED((N2, TW), jnp.float32),
        pltpu.VMEM((C, H), jnp.float32),
        pltpu.VMEM((C, TW), jnp.float32),
        pltpu.VMEM((C,), jnp.int32),
        pltpu.VMEM((ZR, H), jnp.float32),
        pltpu.VMEM((NPT, TW), jnp.float32),
    ],
    compiler_params=pltpu.CompilerParams(needs_layout_passes=False,
                                         use_tc_tiling_on_sc=False),
  )


def _sc_scatter(m_hbm, t_hbm, dst_hbm, agg_out, dpos_out,
                agg_sh, dpos_sh, m_v, t_v, id_v, zb_v, zbt_v):
    cid = lax.axis_index("c")
    sid = lax.axis_index("s")
    wid = sid * NC + cid
    r0 = sid * NPT

    zero = jnp.zeros((L,), jnp.float32)

    def zrow(r, carry):
        for j in range(H // L):
            zb_v[r, pl.ds(j * L, L)] = zero
        return carry

    lax.fori_loop(0, ZR, zrow, 0)

    def zrow_t(r, carry):
        zbt_v[r] = zero
        return carry

    lax.fori_loop(0, NPT, zrow_t, 0)

    for k in range(NPT // ZR):
        pltpu.sync_copy(zb_v, agg_sh.at[pl.ds(r0 + k * ZR, ZR)])
    pltpu.sync_copy(zbt_v, dpos_sh.at[pl.ds(r0, NPT)])
    plsc.subcore_barrier()

    def body(i, carry):
        base = wid * EW + i * C
        pltpu.sync_copy(dst_hbm.at[pl.ds(base, C)], id_v)
        pltpu.sync_copy(m_hbm.at[pl.ds(base, C)], m_v)
        pltpu.sync_copy(t_hbm.at[pl.ds(base, C)], t_v)
        pltpu.sync_copy(m_v, agg_sh.at[id_v], add=True)
        pltpu.sync_copy(t_v, dpos_sh.at[id_v], add=True)
        return carry

    lax.fori_loop(0, NCH, body, 0)
    plsc.subcore_barrier()

    pltpu.sync_copy(agg_sh.at[pl.ds(r0, NPT)], agg_out.at[cid, pl.ds(r0, NPT)])
    pltpu.sync_copy(dpos_sh.at[pl.ds(r0, NPT)], dpos_out.at[cid, pl.ds(r0, NPT)])


def _edge_mlp_body(hs_ref, hd_ref, ea_ref, g_ref,
                   w1s_ref, w1d_ref, w1e_ref, w1g_ref, b1_ref,
                   w2_ref, b2_ref, wc1_ref, bc1_ref, wc2_ref, bc2_ref,
                   m_ref, t_ref):
    f32 = jnp.float32
    bf16 = jnp.bfloat16
    t1 = jnp.dot(hs_ref[...].astype(bf16), w1s_ref[...],
                 preferred_element_type=f32)
    t1 += jnp.dot(hd_ref[...].astype(bf16), w1d_ref[...],
                  preferred_element_type=f32)
    t1 += jnp.dot(ea_ref[...], w1e_ref[...], preferred_element_type=f32)
    d2 = g_ref[:, 3:4]
    t1 += d2 * w1g_ref[...] + b1_ref[...]
    t1 = jnp.maximum(t1, 0.0)
    m = jnp.maximum(jnp.dot(t1.astype(bf16), w2_ref[...],
                            preferred_element_type=f32) + b2_ref[...], 0.0)
    c = jnp.maximum(jnp.dot(m.astype(bf16), wc1_ref[...],
                            preferred_element_type=f32) + bc1_ref[...], 0.0)
    s = jnp.sum(c * wc2_ref[...], axis=1, keepdims=True) + bc2_ref[...]
    coef = jnp.tanh(s) * COORD_SCALE
    inv = lax.rsqrt(d2 + 1e-8)
    dirs = g_ref[:, 0:3] * inv
    m_ref[...] = m
    t_ref[...] = jnp.pad(dirs * coef, ((0, 0), (0, TW - 3)))


def _edge_mlp(hs, hd, ea, geom, w1s, w1d, w1e, w1g, b1, w2, b2, wc1, bc1, wc2, bc2):
    B = 512
    grid = (E // B,)
    full = lambda shape: pl.BlockSpec(shape, lambda i: (0,) * len(shape))
    row = lambda width: pl.BlockSpec((B, width), lambda i: (i, 0))
    return pl.pallas_call(
        _edge_mlp_body,
        grid=grid,
        in_specs=[row(H), row(H), row(DE), row(4),
                  full((H, H)), full((H, H)), full((DE, H)), full((1, H)),
                  full((1, H)), full((H, H)), full((1, H)), full((H, H)),
                  full((1, H)), full((1, H)), full((1, 1))],
        out_specs=[row(H), row(TW)],
        out_shape=[jax.ShapeDtypeStruct((E, H), jnp.float32),
                   jax.ShapeDtypeStruct((E, TW), jnp.float32)],
    )(hs, hd, ea, geom, w1s, w1d, w1e, w1g, b1, w2, b2, wc1, bc1, wc2, bc2)


def _node_body(h_ref, pos_ref, agg_ref, dpos_ref,
               wnh_ref, wna_ref, bn_ref, g_ref, b_ref, ho_ref, po_ref):
    f32 = jnp.float32
    h = h_ref[...]
    agg = agg_ref[0] + agg_ref[1]
    u = jnp.dot(h, wnh_ref[...], preferred_element_type=f32)
    u += jnp.dot(agg, wna_ref[...], preferred_element_type=f32)
    u = jnp.maximum(u + bn_ref[...], 0.0)
    y = h + u
    mu = jnp.mean(y, axis=1, keepdims=True)
    yc = y - mu
    var = jnp.mean(yc * yc, axis=1, keepdims=True)
    ho_ref[...] = yc * lax.rsqrt(var + 1e-5) * g_ref[...] + b_ref[...]
    dp = dpos_ref[0, :, 0:3] + dpos_ref[1, :, 0:3]
    po_ref[...] = pos_ref[...] + dp


def _node_update(h, pos, aggp, dposp, wnh, wna, bn, gamma, beta):
    B = 1000
    grid = (N // B,)
    full = lambda shape: pl.BlockSpec(shape, lambda i: (0,) * len(shape))
    return pl.pallas_call(
        _node_body,
        grid=grid,
        in_specs=[pl.BlockSpec((B, H), lambda i: (i, 0)),
                  pl.BlockSpec((B, 3), lambda i: (i, 0)),
                  pl.BlockSpec((NC, B, H), lambda i: (0, i, 0)),
                  pl.BlockSpec((NC, B, TW), lambda i: (0, i, 0)),
                  full((H, H)), full((H, H)), full((1, H)),
                  full((1, H)), full((1, H))],
        out_specs=[pl.BlockSpec((B, H), lambda i: (i, 0)),
                   pl.BlockSpec((B, 3), lambda i: (i, 0))],
        out_shape=[jax.ShapeDtypeStruct((N, H), jnp.float32),
                   jax.ShapeDtypeStruct((N, 3), jnp.float32)],
    )(h, pos, aggp, dposp, wnh, wna, bn, gamma, beta)


def kernel(h, pos, edge_attr, We1, be1, We2, be2, Wc1, bc1, Wc2, bc2,
           Wn, bn, gamma, beta, edge_index):
    src = edge_index[0]
    dst = edge_index[1]

    hs, hd, geom = _sc_gather_kernel()(h, pos[:, 0], pos[:, 1], pos[:, 2],
                                       src, dst)

    w1s = We1[:, :H].T
    w1d = We1[:, H:2 * H].T
    w1g = We1[:, 2 * H:2 * H + 1].T
    w1e = We1[:, 2 * H + 1:].T
    bf16 = jnp.bfloat16
    m, trans = _edge_mlp(hs, hd, edge_attr.astype(bf16), geom,
                         w1s.astype(bf16), w1d.astype(bf16),
                         w1e.astype(bf16), w1g, be1.reshape(1, H),
                         We2.T.astype(bf16), be2.reshape(1, H),
                         Wc1.T.astype(bf16), bc1.reshape(1, H),
                         Wc2.reshape(1, H), bc2.reshape(1, 1))

    aggp, dposp = _sc_scatter_kernel()(m, trans, dst)
    aggp = aggp[:, :N]
    dposp = dposp[:, :N]

    h_out, pos_out = _node_update(h, pos, aggp, dposp,
                                  Wn[:, :H].T, Wn[:, H:].T, bn.reshape(1, H),
                                  gamma.reshape(1, H), beta.reshape(1, H))
    return (h_out, pos_out)


# speedup vs baseline: 1.5116x; 1.1547x over previous
import functools

import jax
import jax.numpy as jnp
from jax import lax
from jax.experimental import pallas as pl
from jax.experimental.pallas import tpu as pltpu
from jax.experimental.pallas import tpu_sc as plsc

N = 10000
E = 320000
H = 128
DE = 16
COORD_SCALE = 0.1

NC = 2
NS = 16
L = 16
NW = NC * NS
EW = E // NW
C = 80
NCH = EW // C
N2 = 10240
NPT = N2 // NS
ZR = 128
TW = 16

@functools.cache
def _mesh():
    return plsc.VectorSubcoreMesh(core_axis_name="c", subcore_axis_name="s",
                                  num_cores=NC, num_subcores=NS)


@functools.cache
def _sc_gather_kernel():
  return pl.kernel(
    _sc_gather,
    out_type=(
        jax.ShapeDtypeStruct((E, H), jnp.float32),
        jax.ShapeDtypeStruct((E, H), jnp.float32),
        jax.ShapeDtypeStruct((E, 4), jnp.float32),
    ),
    mesh=_mesh(),
    scratch_types=[
        pltpu.VMEM((N,), jnp.float32),
        pltpu.VMEM((N,), jnp.float32),
        pltpu.VMEM((N,), jnp.float32),
        pltpu.VMEM((C,), jnp.int32),
        pltpu.VMEM((C,), jnp.int32),
        pltpu.VMEM((C, H), jnp.float32),
        pltpu.VMEM((C, H), jnp.float32),
        pltpu.VMEM((C, 4), jnp.float32),
        pltpu.SemaphoreType.DMA,
        pltpu.SemaphoreType.DMA,
    ],
    compiler_params=pltpu.CompilerParams(needs_layout_passes=False,
                                         use_tc_tiling_on_sc=False),
  )


def _sc_gather(h_hbm, px_hbm, py_hbm, pz_hbm, src_hbm, dst_hbm, hs_out, hd_out, geom_out,
               px_v, py_v, pz_v, is_v, id_v, hs_v, hd_v, geom_v, sem_s, sem_d):
    cid = lax.axis_index("c")
    sid = lax.axis_index("s")
    wid = sid * NC + cid

    pltpu.sync_copy(px_hbm, px_v)
    pltpu.sync_copy(py_hbm, py_v)
    pltpu.sync_copy(pz_hbm, pz_v)

    def body(i, carry):
        base = wid * EW + i * C
        pltpu.sync_copy(src_hbm.at[pl.ds(base, C)], is_v)
        pltpu.sync_copy(dst_hbm.at[pl.ds(base, C)], id_v)
        cp_s = pltpu.async_copy(h_hbm.at[is_v], hs_v, sem_s)
        cp_d = pltpu.async_copy(h_hbm.at[id_v], hd_v, sem_d)
        for j in range(C // L):
            rs = is_v[pl.ds(j * L, L)]
            rd = id_v[pl.ds(j * L, L)]
            dx = plsc.load_gather(px_v, [rd]) - plsc.load_gather(px_v, [rs])
            dy = plsc.load_gather(py_v, [rd]) - plsc.load_gather(py_v, [rs])
            dz = plsc.load_gather(pz_v, [rd]) - plsc.load_gather(pz_v, [rs])
            d2 = jnp.minimum(dx * dx + dy * dy + dz * dz, 1000.0)
            rows = j * L + lax.iota(jnp.int32, L)
            plsc.store_scatter(geom_v, [rows, jnp.full((L,), 0, jnp.int32)], dx)
            plsc.store_scatter(geom_v, [rows, jnp.full((L,), 1, jnp.int32)], dy)
            plsc.store_scatter(geom_v, [rows, jnp.full((L,), 2, jnp.int32)], dz)
            plsc.store_scatter(geom_v, [rows, jnp.full((L,), 3, jnp.int32)], d2)
        cp_s.wait()
        cp_d.wait()
        pltpu.sync_copy(hs_v, hs_out.at[pl.ds(base, C)])
        pltpu.sync_copy(hd_v, hd_out.at[pl.ds(base, C)])
        pltpu.sync_copy(geom_v, geom_out.at[pl.ds(base, C)])
        return carry

    lax.fori_loop(0, NCH, body, 0)


@functools.cache
def _sc_scatter_kernel():
  return pl.kernel(
    _sc_scatter,
    out_type=(
        jax.ShapeDtypeStruct((NC, N2, H), jnp.float32),
        jax.ShapeDtypeStruct((NC, N2, TW), jnp.float32),
    ),
    mesh=_mesh(),
    scratch_types=[
        pltpu.VMEM_SHARED((N2, H), jnp.float32),
        pltpu.VMEM_SHARED((N2, TW), jnp.float32),
        pltpu.VMEM((C, H), jnp.float32),
        pltpu.VMEM((C, TW), jnp.float32),
        pltpu.VMEM((C,), jnp.int32),
        pltpu.VMEM((ZR, H), jnp.float32),
        pltpu.VMEM((NPT, TW), jnp.float32),
    ],
    compiler_params=pltpu.CompilerParams(needs_layout_passes=False,
                                         use_tc_tiling_on_sc=False),
  )


def _sc_scatter(m_hbm, t_hbm, dst_hbm, agg_out, dpos_out,
                agg_sh, dpos_sh, m_v, t_v, id_v, zb_v, zbt_v):
    cid = lax.axis_index("c")
    sid = lax.axis_index("s")
    wid = sid * NC + cid
    r0 = sid * NPT

    zero = jnp.zeros((L,), jnp.float32)

    def zrow(r, carry):
        for j in range(H // L):
            zb_v[r, pl.ds(j * L, L)] = zero
        return carry

    lax.fori_loop(0, ZR, zrow, 0)

    def zrow_t(r, carry):
        zbt_v[r] = zero
        return carry

    lax.fori_loop(0, NPT, zrow_t, 0)

    for k in range(NPT // ZR):
        pltpu.sync_copy(zb_v, agg_sh.at[pl.ds(r0 + k * ZR, ZR)])
    pltpu.sync_copy(zbt_v, dpos_sh.at[pl.ds(r0, NPT)])
    plsc.subcore_barrier()

    def body(i, carry):
        base = wid * EW + i * C
        pltpu.sync_copy(dst_hbm.at[pl.ds(base, C)], id_v)
        pltpu.sync_copy(m_hbm.at[pl.ds(base, C)], m_v)
        pltpu.sync_copy(t_hbm.at[pl.ds(base, C)], t_v)
        pltpu.sync_copy(m_v, agg_sh.at[id_v], add=True)
        pltpu.sync_copy(t_v, dpos_sh.at[id_v], add=True)
        return carry

    lax.fori_loop(0, NCH, body, 0)
    plsc.subcore_barrier()

    pltpu.sync_copy(agg_sh.at[pl.ds(r0, NPT)], agg_out.at[cid, pl.ds(r0, NPT)])
    pltpu.sync_copy(dpos_sh.at[pl.ds(r0, NPT)], dpos_out.at[cid, pl.ds(r0, NPT)])


def _edge_mlp_body(hs_ref, hd_ref, ea_ref, g_ref,
                   w1s_ref, w1d_ref, w1e_ref, w1g_ref, b1_ref,
                   w2_ref, b2_ref, wc1_ref, bc1_ref, wc2_ref, bc2_ref,
                   m_ref, t_ref):
    f32 = jnp.float32
    bf16 = jnp.bfloat16
    t1 = jnp.dot(hs_ref[...].astype(bf16), w1s_ref[...],
                 preferred_element_type=f32)
    t1 += jnp.dot(hd_ref[...].astype(bf16), w1d_ref[...],
                  preferred_element_type=f32)
    t1 += jnp.dot(ea_ref[...], w1e_ref[...], preferred_element_type=f32)
    d2 = g_ref[:, 3:4]
    t1 += d2 * w1g_ref[...] + b1_ref[...]
    t1 = jnp.maximum(t1, 0.0)
    m = jnp.maximum(jnp.dot(t1.astype(bf16), w2_ref[...],
                            preferred_element_type=f32) + b2_ref[...], 0.0)
    c = jnp.maximum(jnp.dot(m.astype(bf16), wc1_ref[...],
                            preferred_element_type=f32) + bc1_ref[...], 0.0)
    s = jnp.sum(c * wc2_ref[...], axis=1, keepdims=True) + bc2_ref[...]
    coef = jnp.tanh(s) * COORD_SCALE
    inv = lax.rsqrt(d2 + 1e-8)
    dirs = g_ref[:, 0:3] * inv
    m_ref[...] = m
    t_ref[...] = jnp.pad(dirs * coef, ((0, 0), (0, TW - 3)))


def _edge_mlp(hs, hd, ea, geom, w1s, w1d, w1e, w1g, b1, w2, b2, wc1, bc1, wc2, bc2):
    B = 6400
    grid = (E // B,)
    full = lambda shape: pl.BlockSpec(shape, lambda i: (0,) * len(shape))
    row = lambda width: pl.BlockSpec((B, width), lambda i: (i, 0))
    return pl.pallas_call(
        _edge_mlp_body,
        grid=grid,
        in_specs=[row(H), row(H), row(DE), row(4),
                  full((H, H)), full((H, H)), full((DE, H)), full((1, H)),
                  full((1, H)), full((H, H)), full((1, H)), full((H, H)),
                  full((1, H)), full((1, H)), full((1, 1))],
        out_specs=[row(H), row(TW)],
        out_shape=[jax.ShapeDtypeStruct((E, H), jnp.float32),
                   jax.ShapeDtypeStruct((E, TW), jnp.float32)],
    )(hs, hd, ea, geom, w1s, w1d, w1e, w1g, b1, w2, b2, wc1, bc1, wc2, bc2)


def _node_body(h_ref, pos_ref, agg_ref, dpos_ref,
               wnh_ref, wna_ref, bn_ref, g_ref, b_ref, ho_ref, po_ref):
    f32 = jnp.float32
    h = h_ref[...]
    agg = agg_ref[0] + agg_ref[1]
    u = jnp.dot(h, wnh_ref[...], preferred_element_type=f32)
    u += jnp.dot(agg, wna_ref[...], preferred_element_type=f32)
    u = jnp.maximum(u + bn_ref[...], 0.0)
    y = h + u
    mu = jnp.mean(y, axis=1, keepdims=True)
    yc = y - mu
    var = jnp.mean(yc * yc, axis=1, keepdims=True)
    ho_ref[...] = yc * lax.rsqrt(var + 1e-5) * g_ref[...] + b_ref[...]
    dp = dpos_ref[0, :, 0:3] + dpos_ref[1, :, 0:3]
    po_ref[...] = pos_ref[...] + dp


def _node_update(h, pos, aggp, dposp, wnh, wna, bn, gamma, beta):
    B = 1000
    grid = (N // B,)
    full = lambda shape: pl.BlockSpec(shape, lambda i: (0,) * len(shape))
    return pl.pallas_call(
        _node_body,
        grid=grid,
        in_specs=[pl.BlockSpec((B, H), lambda i: (i, 0)),
                  pl.BlockSpec((B, 3), lambda i: (i, 0)),
                  pl.BlockSpec((NC, B, H), lambda i: (0, i, 0)),
                  pl.BlockSpec((NC, B, TW), lambda i: (0, i, 0)),
                  full((H, H)), full((H, H)), full((1, H)),
                  full((1, H)), full((1, H))],
        out_specs=[pl.BlockSpec((B, H), lambda i: (i, 0)),
                   pl.BlockSpec((B, 3), lambda i: (i, 0))],
        out_shape=[jax.ShapeDtypeStruct((N, H), jnp.float32),
                   jax.ShapeDtypeStruct((N, 3), jnp.float32)],
    )(h, pos, aggp, dposp, wnh, wna, bn, gamma, beta)


def kernel(h, pos, edge_attr, We1, be1, We2, be2, Wc1, bc1, Wc2, bc2,
           Wn, bn, gamma, beta, edge_index):
    src = edge_index[0]
    dst = edge_index[1]

    hs, hd, geom = _sc_gather_kernel()(h, pos[:, 0], pos[:, 1], pos[:, 2],
                                       src, dst)

    w1s = We1[:, :H].T
    w1d = We1[:, H:2 * H].T
    w1g = We1[:, 2 * H:2 * H + 1].T
    w1e = We1[:, 2 * H + 1:].T
    bf16 = jnp.bfloat16
    m, trans = _edge_mlp(hs, hd, edge_attr.astype(bf16), geom,
                         w1s.astype(bf16), w1d.astype(bf16),
                         w1e.astype(bf16), w1g, be1.reshape(1, H),
                         We2.T.astype(bf16), be2.reshape(1, H),
                         Wc1.T.astype(bf16), bc1.reshape(1, H),
                         Wc2.reshape(1, H), bc2.reshape(1, 1))

    aggp, dposp = _sc_scatter_kernel()(m, trans, dst)
    aggp = aggp[:, :N]
    dposp = dposp[:, :N]

    h_out, pos_out = _node_update(h, pos, aggp, dposp,
                                  Wn[:, :H].T, Wn[:, H:].T, bn.reshape(1, H),
                                  gamma.reshape(1, H), beta.reshape(1, H))
    return (h_out, pos_out)
